# baseline (speedup 1.0000x reference)
import functools

import jax
import jax.numpy as jnp
from jax.experimental import pallas as pl

_N = 100000
_D = 512
_BN = 2000


def _swap_block(mask_ref, x_ref, xt_ref, u_ref, ut_ref):
    m = mask_ref[:]
    x = x_ref[:]
    xt = xt_ref[:]
    u_ref[:] = jnp.where(m, xt, x)
    ut_ref[:] = jnp.where(m, x, xt)


@functools.partial(jax.jit, static_argnames=())
def kernel(x, x_tilde):
    n, d = x.shape
    bool_swap = jax.random.bernoulli(jax.random.key(42), 0.5, (d,))
    mask = bool_swap[None, :]
    grid = (n // _BN,)
    out = pl.pallas_call(
        _swap_block,
        grid=grid,
        in_specs=[
            pl.BlockSpec((1, d), lambda i: (0, 0)),
            pl.BlockSpec((_BN, d), lambda i: (i, 0)),
            pl.BlockSpec((_BN, d), lambda i: (i, 0)),
        ],
        out_specs=[
            pl.BlockSpec((_BN, d), lambda i: (i, 0)),
            pl.BlockSpec((_BN, d), lambda i: (i, 0)),
        ],
        out_shape=[
            jax.ShapeDtypeStruct((n, d), x.dtype),
            jax.ShapeDtypeStruct((n, d), x.dtype),
        ],
    )(mask, x, x_tilde)
    return (out[0], out[1])
